# Initial kernel scaffold; baseline (speedup 1.0000x reference)
import jax
import jax.numpy as jnp
from jax.experimental import pallas as pl


def kernel(h, psi_x, W1, b1, W2, b2, Wg, bg, mu):
    raise NotImplementedError("write your pallas kernel here")


# speedup vs baseline: 2.0045x; 2.0045x over previous
import functools

import jax
import jax.numpy as jnp
from jax import lax
from jax.experimental import pallas as pl
from jax.experimental.pallas import tpu as pltpu

_B, _S, _D, _E, _K = 4, 2048, 4096, 64, 8
_MIN, _MH, _MOUT = 2, 16, 8
_N = _B * _S
_BT = 512
_LANES = 128


def _router_body(h_ref, psi_ref, w1t_ref, b1_ref, w2t_ref, b2_ref,
                 wgh_ref, wgm_ref, bg_ref, gate_ref, idx_ref):
    f32 = jnp.float32
    bf16 = jnp.bfloat16
    m1 = jnp.dot(psi_ref[...], w1t_ref[...],
                 preferred_element_type=f32) + b1_ref[...]
    m1 = 0.5 * m1 * (1.0 + lax.erf(m1 * (2.0 ** -0.5)))
    m_emb = jnp.dot(m1.astype(bf16), w2t_ref[...],
                    preferred_element_type=f32) + b2_ref[...]
    logits = (jnp.dot(h_ref[...].astype(bf16), wgh_ref[...],
                      preferred_element_type=f32)
              + jnp.dot(m_emb.astype(bf16), wgm_ref[...],
                        preferred_element_type=f32)
              + bg_ref[...])
    lane = lax.broadcasted_iota(jnp.int32, (_BT, _LANES), 1)
    neg = jnp.float32(-jnp.inf)
    logits = jnp.where(lane < _E, logits, neg)

    run = logits
    sel = jnp.zeros((_BT, _LANES), dtype=jnp.bool_)
    idx_acc = jnp.zeros((_BT, _LANES), dtype=jnp.int32)
    for j in range(_K):
        m = jnp.max(run, axis=1, keepdims=True)
        cand = jnp.where(run == m, lane, _LANES)
        aj = jnp.min(cand, axis=1, keepdims=True)
        hit = lane == aj
        sel = jnp.logical_or(sel, hit)
        run = jnp.where(hit, neg, run)
        idx_acc = jnp.where(lane == j, aj, idx_acc)

    mx = jnp.max(logits, axis=1, keepdims=True)
    e = jnp.where(sel, jnp.exp(logits - mx), 0.0)
    gate_ref[...] = e / jnp.sum(e, axis=1, keepdims=True)
    idx_ref[...] = idx_acc


@functools.partial(jax.jit, static_argnames=("interpret",))
def kernel(h, psi_x, W1, b1, W2, b2, Wg, bg, mu, *, interpret=False):
    f32 = jnp.float32
    bf16 = jnp.bfloat16
    hf = h.reshape(_N, _D)
    psi_p = jnp.pad(psi_x.reshape(_N, _MIN),
                    ((0, 0), (0, _LANES - _MIN))).astype(bf16)
    w1t = jnp.pad(W1.T, ((0, _LANES - _MIN), (0, _LANES - _MH))).astype(bf16)
    b1p = jnp.pad(b1, (0, _LANES - _MH)).reshape(1, _LANES)
    w2t = jnp.pad(W2.T, ((0, _LANES - _MH), (0, _LANES - _MOUT))).astype(bf16)
    b2p = jnp.pad(b2, (0, _LANES - _MOUT)).reshape(1, _LANES)
    wgh = jnp.pad(Wg[:, :_D].T, ((0, 0), (0, _LANES - _E))).astype(bf16)
    wgm = jnp.pad(Wg[:, _D:].T,
                  ((0, _LANES - _MOUT), (0, _LANES - _E))).astype(bf16)
    bgp = jnp.pad(bg, (0, _LANES - _E)).reshape(1, _LANES)

    grid = (_N // _BT,)
    tok = lambda i: (i, 0)
    rep = lambda i: (0, 0)
    gate_p, idx_p = pl.pallas_call(
        _router_body,
        grid=grid,
        in_specs=[
            pl.BlockSpec((_BT, _D), tok),
            pl.BlockSpec((_BT, _LANES), tok),
            pl.BlockSpec((_LANES, _LANES), rep),
            pl.BlockSpec((1, _LANES), rep),
            pl.BlockSpec((_LANES, _LANES), rep),
            pl.BlockSpec((1, _LANES), rep),
            pl.BlockSpec((_D, _LANES), rep),
            pl.BlockSpec((_LANES, _LANES), rep),
            pl.BlockSpec((1, _LANES), rep),
        ],
        out_specs=[
            pl.BlockSpec((_BT, _LANES), tok),
            pl.BlockSpec((_BT, _LANES), tok),
        ],
        out_shape=[
            jax.ShapeDtypeStruct((_N, _LANES), f32),
            jax.ShapeDtypeStruct((_N, _LANES), jnp.int32),
        ],
        interpret=interpret,
    )(hf, psi_p, w1t, b1p, w2t, b2p, wgh, wgm, bgp)

    gating = gate_p[:, :_E].reshape(_B, _S, _E)
    indices = idx_p[:, :_K].reshape(_B, _S, _K)
    return gating, indices, mu


# speedup vs baseline: 2.2311x; 1.1130x over previous
import functools

import jax
import jax.numpy as jnp
from jax import lax
from jax.experimental import pallas as pl
from jax.experimental.pallas import tpu as pltpu

_B, _S, _D, _E, _K = 4, 2048, 4096, 64, 8
_MIN, _MH, _MOUT = 2, 16, 8
_N = _B * _S
_BT = 1024
_LANES = 128


def _router_body(h_ref, psi_ref, w1t_ref, b1_ref, w2t_ref, b2_ref,
                 wgh_ref, wgm_ref, bg_ref, gate_ref, idx_ref):
    f32 = jnp.float32
    bf16 = jnp.bfloat16
    m1 = jnp.dot(psi_ref[...], w1t_ref[...],
                 preferred_element_type=f32) + b1_ref[...]
    m1 = 0.5 * m1 * (1.0 + lax.erf(m1 * (2.0 ** -0.5)))
    m_emb = jnp.dot(m1.astype(bf16), w2t_ref[...],
                    preferred_element_type=f32) + b2_ref[...]
    logits = (jnp.dot(h_ref[...].astype(bf16), wgh_ref[...],
                      preferred_element_type=f32)
              + jnp.dot(m_emb.astype(bf16), wgm_ref[...],
                        preferred_element_type=f32)
              + bg_ref[...])
    lane = lax.broadcasted_iota(jnp.int32, (_BT, _LANES), 1)
    neg = jnp.float32(-jnp.inf)
    logits = jnp.where(lane < _E, logits, neg)

    run = logits
    sel = jnp.zeros((_BT, _LANES), dtype=jnp.bool_)
    idx_acc = jnp.zeros((_BT, _LANES), dtype=jnp.int32)
    for j in range(_K):
        m = jnp.max(run, axis=1, keepdims=True)
        cand = jnp.where(run == m, lane, _LANES)
        aj = jnp.min(cand, axis=1, keepdims=True)
        hit = lane == aj
        sel = jnp.logical_or(sel, hit)
        run = jnp.where(hit, neg, run)
        idx_acc = jnp.where(lane == j, aj, idx_acc)

    mx = jnp.max(logits, axis=1, keepdims=True)
    e = jnp.where(sel, jnp.exp(logits - mx), 0.0)
    gate_ref[...] = e / jnp.sum(e, axis=1, keepdims=True)
    idx_ref[...] = idx_acc


@functools.partial(jax.jit, static_argnames=("interpret",))
def kernel(h, psi_x, W1, b1, W2, b2, Wg, bg, mu, *, interpret=False):
    f32 = jnp.float32
    bf16 = jnp.bfloat16
    hf = h.reshape(_N, _D)
    psi_p = jnp.pad(psi_x.reshape(_N, _MIN),
                    ((0, 0), (0, _LANES - _MIN))).astype(bf16)
    w1t = jnp.pad(W1.T, ((0, _LANES - _MIN), (0, _LANES - _MH))).astype(bf16)
    b1p = jnp.pad(b1, (0, _LANES - _MH)).reshape(1, _LANES)
    w2t = jnp.pad(W2.T, ((0, _LANES - _MH), (0, _LANES - _MOUT))).astype(bf16)
    b2p = jnp.pad(b2, (0, _LANES - _MOUT)).reshape(1, _LANES)
    wgh = jnp.pad(Wg[:, :_D].T, ((0, 0), (0, _LANES - _E))).astype(bf16)
    wgm = jnp.pad(Wg[:, _D:].T,
                  ((0, _LANES - _MOUT), (0, _LANES - _E))).astype(bf16)
    bgp = jnp.pad(bg, (0, _LANES - _E)).reshape(1, _LANES)

    grid = (_N // _BT,)
    tok = lambda i: (i, 0)
    rep = lambda i: (0, 0)
    gate_p, idx_p = pl.pallas_call(
        _router_body,
        grid=grid,
        in_specs=[
            pl.BlockSpec((_BT, _D), tok),
            pl.BlockSpec((_BT, _LANES), tok),
            pl.BlockSpec((_LANES, _LANES), rep),
            pl.BlockSpec((1, _LANES), rep),
            pl.BlockSpec((_LANES, _LANES), rep),
            pl.BlockSpec((1, _LANES), rep),
            pl.BlockSpec((_D, _LANES), rep),
            pl.BlockSpec((_LANES, _LANES), rep),
            pl.BlockSpec((1, _LANES), rep),
        ],
        out_specs=[
            pl.BlockSpec((_BT, _LANES), tok),
            pl.BlockSpec((_BT, _LANES), tok),
        ],
        out_shape=[
            jax.ShapeDtypeStruct((_N, _LANES), f32),
            jax.ShapeDtypeStruct((_N, _LANES), jnp.int32),
        ],
        compiler_params=pltpu.CompilerParams(
            dimension_semantics=("arbitrary",)),
        interpret=interpret,
    )(hf, psi_p, w1t, b1p, w2t, b2p, wgh, wgm, bgp)

    gating = gate_p[:, :_E].reshape(_B, _S, _E)
    indices = idx_p[:, :_K].reshape(_B, _S, _K)
    return gating, indices, mu
